# initial kernel scaffold (unmeasured)
import jax
import jax.numpy as jnp
from jax import lax
from jax.experimental import pallas as pl
from jax.experimental.pallas import tpu as pltpu

N_DEV = 32


def kernel(x, w_mat):
    m_per, k = x.shape
    _, n_per = w_mat.shape
    m_total = N_DEV * m_per

    def body(x_ref, w_ref, out_ref, xfull_ref, send_sem, recv_sems):
        my = lax.axis_index("i")
        left = lax.rem(my + N_DEV - 1, N_DEV)
        right = lax.rem(my + 1, N_DEV)

        barrier = pltpu.get_barrier_semaphore()
        pl.semaphore_signal(barrier, inc=1, device_id=(left,),
                            device_id_type=pl.DeviceIdType.MESH)
        pl.semaphore_signal(barrier, inc=1, device_id=(right,),
                            device_id_type=pl.DeviceIdType.MESH)
        pl.semaphore_wait(barrier, 2)

        xfull_ref[pl.ds(my * m_per, m_per), :] = x_ref[...]

        for h in range(1, N_DEV):
            o_send = lax.rem(my - (h - 1) + N_DEV, N_DEV)
            o_recv = lax.rem(my - h + N_DEV, N_DEV)
            send = pltpu.make_async_remote_copy(
                src_ref=xfull_ref.at[pl.ds(o_send * m_per, m_per), :],
                dst_ref=xfull_ref.at[pl.ds(o_send * m_per, m_per), :],
                send_sem=send_sem,
                recv_sem=recv_sems.at[h],
                device_id=(right,),
                device_id_type=pl.DeviceIdType.MESH,
            )
            send.start()
            recv = pltpu.make_async_remote_copy(
                src_ref=xfull_ref.at[pl.ds(o_recv * m_per, m_per), :],
                dst_ref=xfull_ref.at[pl.ds(o_recv * m_per, m_per), :],
                send_sem=send_sem,
                recv_sem=recv_sems.at[h],
                device_id=(left,),
                device_id_type=pl.DeviceIdType.MESH,
            )
            send.wait_send()
            recv.wait_recv()

        out_ref[...] = jnp.dot(xfull_ref[...], w_ref[...],
                               preferred_element_type=jnp.float32)

    return pl.pallas_call(
        body,
        out_shape=jax.ShapeDtypeStruct((m_total, n_per), jnp.float32),
        in_specs=[
            pl.BlockSpec(memory_space=pltpu.VMEM),
            pl.BlockSpec(memory_space=pltpu.VMEM),
        ],
        out_specs=pl.BlockSpec(memory_space=pltpu.VMEM),
        scratch_shapes=[
            pltpu.VMEM((m_total, k), jnp.float32),
            pltpu.SemaphoreType.DMA,
            pltpu.SemaphoreType.DMA((N_DEV,)),
        ],
        compiler_params=pltpu.CompilerParams(collective_id=0),
    )(x, w_mat)


# baseline (device time: 245558 ns/iter reference)
import jax
import jax.numpy as jnp
from jax import lax
from jax.experimental import pallas as pl
from jax.experimental.pallas import tpu as pltpu

N_DEV = 32


def kernel(x, w_mat):
    m_per, k = x.shape
    _, n_per = w_mat.shape
    m_total = N_DEV * m_per

    def body(x_ref, w_ref, out_ref, comm_ref, send_sem, recv_sems):
        my = lax.axis_index("i")
        left = lax.rem(my + N_DEV - 1, N_DEV)
        right = lax.rem(my + 1, N_DEV)

        barrier = pltpu.get_barrier_semaphore()
        pl.semaphore_signal(barrier, inc=1, device_id=(left,),
                            device_id_type=pl.DeviceIdType.MESH)
        pl.semaphore_signal(barrier, inc=1, device_id=(right,),
                            device_id_type=pl.DeviceIdType.MESH)
        pl.semaphore_wait(barrier, 2)

        comm_ref[0, :, :] = x_ref[...]
        out_ref[pl.ds(my * m_per, m_per), :] = jnp.dot(
            x_ref[...], w_ref[...], preferred_element_type=jnp.float32)

        for h in range(1, N_DEV):
            send = pltpu.make_async_remote_copy(
                src_ref=comm_ref.at[h - 1],
                dst_ref=comm_ref.at[h],
                send_sem=send_sem,
                recv_sem=recv_sems.at[h],
                device_id=(right,),
                device_id_type=pl.DeviceIdType.MESH,
            )
            send.start()
            send.wait_send()
            recv = pltpu.make_async_remote_copy(
                src_ref=comm_ref.at[h - 1],
                dst_ref=comm_ref.at[h],
                send_sem=send_sem,
                recv_sem=recv_sems.at[h],
                device_id=(left,),
                device_id_type=pl.DeviceIdType.MESH,
            )
            recv.wait_recv()
            origin = lax.rem(my - h + N_DEV, N_DEV)
            out_ref[pl.ds(origin * m_per, m_per), :] = jnp.dot(
                comm_ref[h], w_ref[...], preferred_element_type=jnp.float32)

    return pl.pallas_call(
        body,
        out_shape=jax.ShapeDtypeStruct((m_total, n_per), jnp.float32),
        in_specs=[
            pl.BlockSpec(memory_space=pltpu.VMEM),
            pl.BlockSpec(memory_space=pltpu.VMEM),
        ],
        out_specs=pl.BlockSpec(memory_space=pltpu.VMEM),
        scratch_shapes=[
            pltpu.VMEM((N_DEV, m_per, k), jnp.float32),
            pltpu.SemaphoreType.DMA,
            pltpu.SemaphoreType.DMA((N_DEV,)),
        ],
        compiler_params=pltpu.CompilerParams(collective_id=0),
    )(x, w_mat)


# device time: 187375 ns/iter; 1.3105x vs baseline; 1.3105x over previous
import jax
import jax.numpy as jnp
from jax import lax
from jax.experimental import pallas as pl
from jax.experimental.pallas import tpu as pltpu

N_DEV = 32
H_R = N_DEV // 2
H_L = N_DEV - 1 - H_R


def kernel(x, w_mat):
    m_per, k = x.shape
    _, n_per = w_mat.shape
    m_total = N_DEV * m_per

    def body(x_ref, w_ref, out_ref, comm_ref,
             send_r_sems, send_l_sems, recv_r_sems, recv_l_sems):
        my = lax.axis_index("i")
        left = lax.rem(my + N_DEV - 1, N_DEV)
        right = lax.rem(my + 1, N_DEV)

        def r_slot(h):
            return h

        def l_slot(h):
            return H_R + h

        def send_to(nbr, src_slot, dst_slot, s_sem, r_sem):
            rdma = pltpu.make_async_remote_copy(
                src_ref=comm_ref.at[src_slot],
                dst_ref=comm_ref.at[dst_slot],
                send_sem=s_sem,
                recv_sem=r_sem,
                device_id=(nbr,),
                device_id_type=pl.DeviceIdType.MESH,
            )
            rdma.start()
            return rdma

        def wait_recv(dst_slot, r_sem, s_sem):
            rdma = pltpu.make_async_remote_copy(
                src_ref=comm_ref.at[dst_slot],
                dst_ref=comm_ref.at[dst_slot],
                send_sem=s_sem,
                recv_sem=r_sem,
                device_id=(left,),
                device_id_type=pl.DeviceIdType.MESH,
            )
            rdma.wait_recv()

        def gemm_chunk(slot, origin):
            out_ref[pl.ds(origin * m_per, m_per), :] = jnp.dot(
                comm_ref[slot], w_ref[...],
                preferred_element_type=jnp.float32)

        barrier = pltpu.get_barrier_semaphore()
        pl.semaphore_signal(barrier, inc=1, device_id=(left,),
                            device_id_type=pl.DeviceIdType.MESH)
        pl.semaphore_signal(barrier, inc=1, device_id=(right,),
                            device_id_type=pl.DeviceIdType.MESH)
        pl.semaphore_wait(barrier, 2)

        comm_ref[0, :, :] = x_ref[...]

        send_to(right, 0, r_slot(1), send_r_sems.at[1], recv_r_sems.at[1])
        send_to(left, 0, l_slot(1), send_l_sems.at[1], recv_l_sems.at[1])
        gemm_chunk(0, my)

        for h in range(1, H_R + 1):
            wait_recv(r_slot(h), recv_r_sems.at[h], send_r_sems.at[h])
            if h + 1 <= H_R:
                send_to(right, r_slot(h), r_slot(h + 1),
                        send_r_sems.at[h + 1], recv_r_sems.at[h + 1])
            if h <= H_L:
                wait_recv(l_slot(h), recv_l_sems.at[h], send_l_sems.at[h])
                if h + 1 <= H_L:
                    send_to(left, l_slot(h), l_slot(h + 1),
                            send_l_sems.at[h + 1], recv_l_sems.at[h + 1])
            gemm_chunk(r_slot(h), lax.rem(my - h + N_DEV, N_DEV))
            if h <= H_L:
                gemm_chunk(l_slot(h), lax.rem(my + h, N_DEV))

        for h in range(1, H_R + 1):
            src = r_slot(h - 1) if h > 1 else 0
            drain = pltpu.make_async_remote_copy(
                src_ref=comm_ref.at[src],
                dst_ref=comm_ref.at[r_slot(h)],
                send_sem=send_r_sems.at[h],
                recv_sem=recv_r_sems.at[h],
                device_id=(right,),
                device_id_type=pl.DeviceIdType.MESH,
            )
            drain.wait_send()
        for h in range(1, H_L + 1):
            src = l_slot(h - 1) if h > 1 else 0
            drain = pltpu.make_async_remote_copy(
                src_ref=comm_ref.at[src],
                dst_ref=comm_ref.at[l_slot(h)],
                send_sem=send_l_sems.at[h],
                recv_sem=recv_l_sems.at[h],
                device_id=(left,),
                device_id_type=pl.DeviceIdType.MESH,
            )
            drain.wait_send()

    return pl.pallas_call(
        body,
        out_shape=jax.ShapeDtypeStruct((m_total, n_per), jnp.float32),
        in_specs=[
            pl.BlockSpec(memory_space=pltpu.VMEM),
            pl.BlockSpec(memory_space=pltpu.VMEM),
        ],
        out_specs=pl.BlockSpec(memory_space=pltpu.VMEM),
        scratch_shapes=[
            pltpu.VMEM((N_DEV, m_per, k), jnp.float32),
            pltpu.SemaphoreType.DMA((H_R + 1,)),
            pltpu.SemaphoreType.DMA((H_L + 1,)),
            pltpu.SemaphoreType.DMA((H_R + 1,)),
            pltpu.SemaphoreType.DMA((H_L + 1,)),
        ],
        compiler_params=pltpu.CompilerParams(collective_id=0),
    )(x, w_mat)
